# Initial kernel scaffold; baseline (speedup 1.0000x reference)
import jax
import jax.numpy as jnp
from jax.experimental import pallas as pl


def kernel(x, edge_index, Wc, bc, bn_g, bn_b, sbn_g, sbn_b, Ws, bs, Wl, plif_w):
    raise NotImplementedError("write your pallas kernel here")


# speedup vs baseline: 1.1958x; 1.1958x over previous
import functools

import jax
import jax.numpy as jnp
from jax.experimental import pallas as pl

_N = 10000
_E = 320000
_D = 128
_H = 128
_T = 32
_CH = _D // _T
_V_TH = 0.005
_EPS = 1e-5

_BN = 1000


def _lif_body(h2_ref, tau_ref, wl_ref, z_ref):
    tau = tau_ref[0, 0]
    wl = wl_ref[0, :]
    v = jnp.zeros((_BN, _H), jnp.float32)
    cols = []
    for t in range(_T):
        v = v + (h2_ref[t] - v) * tau
        s = (v >= _V_TH).astype(jnp.float32)
        v = v * (1.0 - s)
        cols.append(jnp.sum(s * wl[None, :], axis=1))
    z_ref[...] = jnp.stack(cols, axis=1)


def _lif_pallas(h2, tau_inv, wl):
    grid = (_N // _BN,)
    return pl.pallas_call(
        _lif_body,
        grid=grid,
        in_specs=[
            pl.BlockSpec((_T, _BN, _H), lambda i: (0, i, 0)),
            pl.BlockSpec((1, 1), lambda i: (0, 0)),
            pl.BlockSpec((1, _H), lambda i: (0, 0)),
        ],
        out_specs=pl.BlockSpec((_BN, _T), lambda i: (i, 0)),
        out_shape=jax.ShapeDtypeStruct((_N, _T), jnp.float32),
    )(h2, tau_inv.reshape(1, 1), wl.reshape(1, _H)).T


def kernel(x, edge_index, Wc, bc, bn_g, bn_b, sbn_g, sbn_b, Ws, bs, Wl, plif_w):
    src = edge_index[0]
    dst = edge_index[1]
    deg = jax.ops.segment_sum(jnp.ones((_E,), jnp.float32), dst,
                              num_segments=_N) + 1.0
    dinv = jax.lax.rsqrt(deg)

    m = jnp.mean(x, axis=0)
    var = jnp.var(x, axis=0)
    g = bn_g.reshape(-1)
    b = bn_b.reshape(-1)
    xb = (x - m) * jax.lax.rsqrt(var + _EPS) * g + b

    def _agg(yt):
        y = yt * dinv[:, None]
        return (jax.ops.segment_sum(y[src], dst, num_segments=_N)
                + y) * dinv[:, None]

    xbr = xb.reshape(_N, _T, _CH).transpose(1, 0, 2)
    p = jnp.einsum('tnc,tch->tnh', xbr, Wc)
    g1 = jax.vmap(_agg)(p) + bc[:, None, :]
    e = jax.nn.elu(g1)
    me = jnp.mean(e, axis=1, keepdims=True)
    ve = jnp.var(e, axis=1, keepdims=True)
    h1 = (e - me) * jax.lax.rsqrt(ve + _EPS) * sbn_g + sbn_b

    q = jnp.einsum('tnh,hk->tnk', h1, Ws)
    h2 = jax.vmap(_agg)(q) + bs

    tau_inv = jax.nn.sigmoid(plif_w)
    wl = Wl.sum(axis=1)
    z = _lif_pallas(h2, tau_inv, wl)
    return jnp.concatenate([z, z], axis=0)


# speedup vs baseline: 6.1718x; 5.1614x over previous
import functools

import jax
import jax.numpy as jnp
from jax import lax
from jax.experimental import pallas as pl
from jax.experimental.pallas import tpu as pltpu
from jax.experimental.pallas import tpu_sc as plsc

_N = 10000
_E = 320000
_D = 128
_H = 128
_T = 32
_CH = _D // _T
_V_TH = 0.005
_EPS = 1e-5

_NTILE = 16
_NP = 10112
_RPT = _NP // _NTILE
_K = 128
_EPT = 20096
_EP = _EPT * _NTILE
_NCHUNK = _EPT // _K


def _make_sc_agg(B):
    b_per_core = max(B // 2, 1)
    mesh = plsc.VectorSubcoreMesh(core_axis_name="c", subcore_axis_name="s")

    @functools.partial(
        pl.kernel,
        out_type=jax.ShapeDtypeStruct((B, _NP, _H), jnp.float32),
        mesh=mesh,
        scratch_types=[
            pltpu.VMEM((_K,), jnp.int32),
            pltpu.VMEM((_K,), jnp.int32),
            pltpu.VMEM((_K, _H), jnp.float32),
            pltpu.VMEM_SHARED((_NP, _H), jnp.float32),
            pltpu.SemaphoreType.DMA,
        ],
    )
    def agg(y_hbm, src_hbm, dst_hbm, out_hbm, srcb, dstb, rows, acc, sem):
        cid = lax.axis_index("c")
        sid = lax.axis_index("s")
        row0 = sid * _RPT
        for bl in range(b_per_core):
            b = cid * b_per_core + bl

            @pl.when(b < B)
            def _body():
                pltpu.sync_copy(y_hbm.at[b, pl.ds(row0, _RPT)],
                                acc.at[pl.ds(row0, _RPT)])
                plsc.subcore_barrier()

                def chunk(i, carry):
                    base = sid * _EPT + i * _K
                    pltpu.sync_copy(src_hbm.at[pl.ds(base, _K)], srcb)
                    pltpu.sync_copy(dst_hbm.at[pl.ds(base, _K)], dstb)
                    pltpu.async_copy(y_hbm.at[b].at[srcb], rows, sem).wait()
                    pltpu.sync_copy(rows, acc.at[dstb], add=True)
                    return carry

                lax.fori_loop(0, _NCHUNK, chunk, 0)
                plsc.subcore_barrier()
                pltpu.sync_copy(acc.at[pl.ds(row0, _RPT)],
                                out_hbm.at[b, pl.ds(row0, _RPT)])

    return agg


_SC_AGG_T = _make_sc_agg(_T)

_BN = 1000


def _lif_body(h2_ref, tau_ref, wl_ref, z_ref):
    tau = tau_ref[0, 0]
    wl = wl_ref[0, :]
    v = jnp.zeros((_BN, _H), jnp.float32)
    cols = []
    for t in range(_T):
        v = v + (h2_ref[t] - v) * tau
        s = (v >= _V_TH).astype(jnp.float32)
        v = v * (1.0 - s)
        cols.append(jnp.sum(s * wl[None, :], axis=1))
    z_ref[...] = jnp.stack(cols, axis=1)


def _lif_pallas(h2, tau_inv, wl):
    grid = (_N // _BN,)
    return pl.pallas_call(
        _lif_body,
        grid=grid,
        in_specs=[
            pl.BlockSpec((_T, _BN, _H), lambda i: (0, i, 0)),
            pl.BlockSpec((1, 1), lambda i: (0, 0)),
            pl.BlockSpec((1, _H), lambda i: (0, 0)),
        ],
        out_specs=pl.BlockSpec((_BN, _T), lambda i: (i, 0)),
        out_shape=jax.ShapeDtypeStruct((_N, _T), jnp.float32),
    )(h2, tau_inv.reshape(1, 1), wl.reshape(1, _H)).T


def kernel(x, edge_index, Wc, bc, bn_g, bn_b, sbn_g, sbn_b, Ws, bs, Wl, plif_w):
    src = edge_index[0]
    dst = edge_index[1]
    deg = jax.ops.segment_sum(jnp.ones((_E,), jnp.float32), dst,
                              num_segments=_N) + 1.0
    dinv = jax.lax.rsqrt(deg)

    m = jnp.mean(x, axis=0)
    var = jnp.var(x, axis=0)
    g = bn_g.reshape(-1)
    b = bn_b.reshape(-1)
    xb = (x - m) * jax.lax.rsqrt(var + _EPS) * g + b

    srcp = jnp.concatenate(
        [src.astype(jnp.int32), jnp.zeros((_EP - _E,), jnp.int32)])
    dstp = jnp.concatenate(
        [dst.astype(jnp.int32), jnp.full((_EP - _E,), _N, jnp.int32)])

    def _agg_batch(pt):
        y = pt * dinv[None, :, None]
        y_pad = jnp.concatenate(
            [y, jnp.zeros((_T, _NP - _N, _H), jnp.float32)], axis=1)
        out = _SC_AGG_T(y_pad, srcp, dstp)
        return out[:, :_N, :] * dinv[None, :, None]

    xbr = xb.reshape(_N, _T, _CH).transpose(1, 0, 2)
    p = jnp.einsum('tnc,tch->tnh', xbr, Wc)
    g1 = _agg_batch(p) + bc[:, None, :]
    e = jax.nn.elu(g1)
    me = jnp.mean(e, axis=1, keepdims=True)
    ve = jnp.var(e, axis=1, keepdims=True)
    h1 = (e - me) * jax.lax.rsqrt(ve + _EPS) * sbn_g + sbn_b

    q = jnp.einsum('tnh,hk->tnk', h1, Ws)
    h2 = _agg_batch(q) + bs

    tau_inv = jax.nn.sigmoid(plif_w)
    wl = Wl.sum(axis=1)
    z = _lif_pallas(h2, tau_inv, wl)
    return jnp.concatenate([z, z], axis=0)
